# baseline (speedup 1.0000x reference)
import jax
import jax.numpy as jnp
from jax import lax
from jax.experimental import pallas as pl
from jax.experimental.pallas import tpu as pltpu

_VMEM_LIMIT = 100 * 1024 * 1024
_EPS = 1e-6


def _stem_kernel(x_ref, w_ref, b_ref, o_ref, st_ref):
    acc = jnp.dot(x_ref[0], w_ref[...],
                  preferred_element_type=jnp.float32) + b_ref[...]
    o_ref[0] = acc.astype(o_ref.dtype)
    s = jnp.sum(acc, axis=0)
    ss = jnp.sum(acc * acc, axis=0)
    st_ref[0] = jnp.concatenate([s[None, :], ss[None, :]], axis=0)


def _scale_shift(s, ss, gm_ref, g_ref, bt_ref, inv):
    mean = jnp.dot(s, gm_ref[...], preferred_element_type=jnp.float32,
                   precision=lax.Precision.HIGHEST) * inv
    ex2 = jnp.dot(ss, gm_ref[...], preferred_element_type=jnp.float32,
                  precision=lax.Precision.HIGHEST) * inv
    var = ex2 - mean * mean
    scale = g_ref[...] * lax.rsqrt(var + _EPS)
    shift = bt_ref[...] - mean * scale
    return scale, shift


def _gn_silu_conv3x3(x, scale, shift, w_ref, cb_ref, S, cin):
    hw = S * S
    y = x * scale + shift
    y = y * jax.nn.sigmoid(y)
    yb = y.astype(jnp.bfloat16).reshape(S, S, cin)
    ypf = jnp.pad(yb, ((1, 2), (1, 1), (0, 0))).reshape((S + 3) * (S + 2), cin)
    mw = S * (S + 2)
    acc_w = None
    for t in range(9):
        dy, dx = divmod(t, 3)
        shift_t = dy * (S + 2) + dx
        part = jnp.dot(ypf[shift_t:shift_t + mw, :],
                       w_ref[t * cin:(t + 1) * cin, :],
                       preferred_element_type=jnp.float32)
        acc_w = part if acc_w is None else acc_w + part
    acc = acc_w.reshape(S, S + 2, -1)[:, :S, :].reshape(hw, -1)
    return acc + cb_ref[...]


def _make_resblock_kernel(S, cin, cout, cg1, cg2, has_proj, do_pool):
    hw = S * S
    inv1 = 1.0 / float(hw * cg1)
    inv2 = 1.0 / float(hw * cg2)

    def body(*refs):
        (x_ref, stin_ref, g1_ref, b1_ref, gm1_ref, w1_ref, cb1_ref,
         g2_ref, b2_ref, gm2_ref, w2_ref, cb2_ref) = refs[:12]
        n = 12
        scw_ref = scb_ref = None
        if has_proj:
            scw_ref, scb_ref = refs[12:14]
            n = 14
        o_ref = refs[n]
        pool_ref = stp_ref = None
        if do_pool:
            pool_ref, stp_ref = refs[n + 1], refs[n + 2]
            h_s, st2_s = refs[n + 3], refs[n + 4]
        else:
            h_s, st2_s = refs[n + 1], refs[n + 2]

        p = pl.program_id(1)

        @pl.when(p == 0)
        def _phase_conv1():
            scale, shift = _scale_shift(stin_ref[0, 0:1, :], stin_ref[0, 1:2, :],
                                        gm1_ref, g1_ref, b1_ref, inv1)
            acc = _gn_silu_conv3x3(x_ref[0].astype(jnp.float32), scale, shift,
                                   w1_ref, cb1_ref, S, cin)
            h_s[...] = acc.astype(h_s.dtype)
            ts = jnp.sum(acc, axis=0)
            tss = jnp.sum(acc * acc, axis=0)
            st2_s[...] = jnp.concatenate([ts[None, :], tss[None, :]], axis=0)

        @pl.when(p == 1)
        def _phase_conv2():
            scale, shift = _scale_shift(st2_s[0:1, :], st2_s[1:2, :],
                                        gm2_ref, g2_ref, b2_ref, inv2)
            acc = _gn_silu_conv3x3(h_s[...].astype(jnp.float32), scale, shift,
                                   w2_ref, cb2_ref, S, cout)
            if has_proj:
                acc = acc + (jnp.dot(x_ref[0], scw_ref[...],
                                     preferred_element_type=jnp.float32)
                             + scb_ref[...])
            else:
                acc = acc + x_ref[0].astype(jnp.float32)
            o_ref[0] = acc.astype(o_ref.dtype)
            if do_pool:
                v = acc.reshape(S // 2, 2, S // 2, 2, cout)
                pq = 0.25 * (v[:, 0, :, 0, :] + v[:, 0, :, 1, :]
                             + v[:, 1, :, 0, :] + v[:, 1, :, 1, :])
                pf = pq.reshape(hw // 4, cout)
                pool_ref[0] = pf.astype(pool_ref.dtype)
                ps = jnp.sum(pf, axis=0)
                pss = jnp.sum(pf * pf, axis=0)
                stp_ref[0] = jnp.concatenate([ps[None, :], pss[None, :]],
                                             axis=0)

    return body


def _resblock(xf, stats, gn1, gm1, w1, cb1, gn2, gm2, w2, cb2, *, S, groups,
              sc=None, do_pool=False):
    B, hw, cin = xf.shape
    cout = w1.shape[-1]
    f32 = jnp.float32

    def _c(i):
        return lambda b, p: (b,) + (0,) * i

    in_specs = [
        pl.BlockSpec((1, hw, cin), lambda b, p: (b, 0, 0)),
        pl.BlockSpec((1, 2, cin), lambda b, p: (b, 0, 0)),
        pl.BlockSpec((1, cin), lambda b, p: (0, 0)),
        pl.BlockSpec((1, cin), lambda b, p: (0, 0)),
        pl.BlockSpec((cin, cin), lambda b, p: (0, 0)),
        pl.BlockSpec((9 * cin, cout), lambda b, p: (0, 0)),
        pl.BlockSpec((1, cout), lambda b, p: (0, 0)),
        pl.BlockSpec((1, cout), lambda b, p: (0, 0)),
        pl.BlockSpec((1, cout), lambda b, p: (0, 0)),
        pl.BlockSpec((cout, cout), lambda b, p: (0, 0)),
        pl.BlockSpec((9 * cout, cout), lambda b, p: (0, 0)),
        pl.BlockSpec((1, cout), lambda b, p: (0, 0)),
    ]
    args = [xf, stats,
            gn1[0].reshape(1, cin).astype(f32), gn1[1].reshape(1, cin).astype(f32),
            gm1, w1, cb1.reshape(1, cout).astype(f32),
            gn2[0].reshape(1, cout).astype(f32), gn2[1].reshape(1, cout).astype(f32),
            gm2, w2, cb2.reshape(1, cout).astype(f32)]
    if sc is not None:
        in_specs += [pl.BlockSpec((cin, cout), lambda b, p: (0, 0)),
                     pl.BlockSpec((1, cout), lambda b, p: (0, 0))]
        args += [sc[0].astype(jnp.bfloat16), sc[1].reshape(1, cout).astype(f32)]

    out_shapes = [jax.ShapeDtypeStruct((B, hw, cout), f32)]
    out_specs = [pl.BlockSpec((1, hw, cout), lambda b, p: (b, 0, 0))]
    if do_pool:
        out_shapes += [jax.ShapeDtypeStruct((B, hw // 4, cout), jnp.bfloat16),
                       jax.ShapeDtypeStruct((B, 2, cout), f32)]
        out_specs += [pl.BlockSpec((1, hw // 4, cout), lambda b, p: (b, 0, 0)),
                      pl.BlockSpec((1, 2, cout), lambda b, p: (b, 0, 0))]

    return pl.pallas_call(
        _make_resblock_kernel(S, cin, cout, cin // groups, cout // groups,
                              sc is not None, do_pool),
        out_shape=tuple(out_shapes),
        grid=(B, 2),
        in_specs=in_specs,
        out_specs=tuple(out_specs),
        scratch_shapes=[pltpu.VMEM((hw, cout), jnp.bfloat16),
                        pltpu.VMEM((2, cout), f32)],
        compiler_params=pltpu.CompilerParams(
            dimension_semantics=("parallel", "arbitrary"),
            vmem_limit_bytes=_VMEM_LIMIT),
    )(*args)


def _group_mat(c, groups):
    gidx = jnp.arange(c) // (c // groups)
    return (gidx[:, None] == gidx[None, :]).astype(jnp.float32)


def kernel(x, conv_in_w, conv_in_b,
           r0_gn1_gamma, r0_gn1_beta, r0_conv1_w, r0_conv1_b,
           r0_gn2_gamma, r0_gn2_beta, r0_conv2_w, r0_conv2_b,
           r1_gn1_gamma, r1_gn1_beta, r1_conv1_w, r1_conv1_b,
           r1_gn2_gamma, r1_gn2_beta, r1_conv2_w, r1_conv2_b,
           r1_sc_w, r1_sc_b,
           r2_gn1_gamma, r2_gn1_beta, r2_conv1_w, r2_conv1_b,
           r2_gn2_gamma, r2_gn2_beta, r2_conv2_w, r2_conv2_b,
           r2_sc_w, r2_sc_b):
    groups = 32
    B, c0, hr, wr = x.shape
    H, W = hr // 2, wr // 2
    cu = c0 * 4
    xu = (x.reshape(B, c0, H, 2, W, 2).transpose(0, 2, 4, 1, 3, 5)
          .reshape(B, H * W, cu).astype(jnp.bfloat16))

    cin0 = conv_in_w.shape[1]
    stem_out, st = pl.pallas_call(
        _stem_kernel,
        out_shape=(jax.ShapeDtypeStruct((B, H * W, cin0), jnp.bfloat16),
                   jax.ShapeDtypeStruct((B, 2, cin0), jnp.float32)),
        grid=(B,),
        in_specs=[pl.BlockSpec((1, H * W, cu), lambda b: (b, 0, 0)),
                  pl.BlockSpec((cu, cin0), lambda b: (0, 0)),
                  pl.BlockSpec((1, cin0), lambda b: (0, 0))],
        out_specs=(pl.BlockSpec((1, H * W, cin0), lambda b: (b, 0, 0)),
                   pl.BlockSpec((1, 2, cin0), lambda b: (b, 0, 0))),
        compiler_params=pltpu.CompilerParams(
            dimension_semantics=("parallel",),
            vmem_limit_bytes=_VMEM_LIMIT),
    )(xu, conv_in_w.astype(jnp.bfloat16),
      conv_in_b.reshape(1, cin0).astype(jnp.float32))

    blocks = [
        dict(gn1=(r0_gn1_gamma, r0_gn1_beta), w1=r0_conv1_w, b1=r0_conv1_b,
             gn2=(r0_gn2_gamma, r0_gn2_beta), w2=r0_conv2_w, b2=r0_conv2_b,
             sc=None),
        dict(gn1=(r1_gn1_gamma, r1_gn1_beta), w1=r1_conv1_w, b1=r1_conv1_b,
             gn2=(r1_gn2_gamma, r1_gn2_beta), w2=r1_conv2_w, b2=r1_conv2_b,
             sc=(r1_sc_w, r1_sc_b)),
        dict(gn1=(r2_gn1_gamma, r2_gn1_beta), w1=r2_conv1_w, b1=r2_conv1_b,
             gn2=(r2_gn2_gamma, r2_gn2_beta), w2=r2_conv2_w, b2=r2_conv2_b,
             sc=(r2_sc_w, r2_sc_b)),
    ]

    feats = []
    cur, cur_st = stem_out, st
    S = H
    gmats = {}
    for i, bp in enumerate(blocks):
        cin = bp["w1"].shape[2]
        cout = bp["w1"].shape[3]
        for c in (cin, cout):
            if c not in gmats:
                gmats[c] = _group_mat(c, groups)
        w1 = bp["w1"].reshape(9 * cin, cout).astype(jnp.bfloat16)
        w2 = bp["w2"].reshape(9 * cout, cout).astype(jnp.bfloat16)
        last = i == len(blocks) - 1
        out = _resblock(
            cur, cur_st, bp["gn1"], gmats[cin], w1, bp["b1"],
            bp["gn2"], gmats[cout], w2, bp["b2"],
            S=S, groups=groups, sc=bp["sc"], do_pool=not last)
        if last:
            feat = out[0]
        else:
            feat, cur, cur_st = out
        feats.append(feat.reshape(B, S, S, cout).transpose(0, 3, 1, 2))
        S //= 2
    return feats


# speedup vs baseline: 1.2471x; 1.2471x over previous
import jax
import jax.numpy as jnp
from jax import lax
from jax.experimental import pallas as pl
from jax.experimental.pallas import tpu as pltpu

_VMEM_LIMIT = 100 * 1024 * 1024
_EPS = 1e-6


def _stem_kernel(x_ref, w_ref, b_ref, o_ref, st_ref):
    acc = jnp.dot(x_ref[0], w_ref[...],
                  preferred_element_type=jnp.float32) + b_ref[...]
    o_ref[0] = acc.astype(o_ref.dtype)
    s = jnp.sum(acc, axis=0)
    ss = jnp.sum(acc * acc, axis=0)
    st_ref[0] = jnp.concatenate([s[None, :], ss[None, :]], axis=0)


def _scale_shift(s, ss, gm_ref, g_ref, bt_ref, inv):
    mean = jnp.dot(s, gm_ref[...], preferred_element_type=jnp.float32,
                   precision=lax.Precision.HIGHEST) * inv
    ex2 = jnp.dot(ss, gm_ref[...], preferred_element_type=jnp.float32,
                  precision=lax.Precision.HIGHEST) * inv
    var = ex2 - mean * mean
    scale = g_ref[...] * lax.rsqrt(var + _EPS)
    shift = bt_ref[...] - mean * scale
    return scale, shift


def _gn_silu_conv3x3(x, scale, shift, w_ref, cb_ref, S, cin):
    hw = S * S
    y = x * scale + shift
    y = y * jax.nn.sigmoid(y)
    yb = y.astype(jnp.bfloat16).reshape(S, S, cin)
    yp = jnp.pad(yb, ((1, 1), (1, 1), (0, 0)))
    patches = jnp.concatenate(
        [yp[dy:dy + S, dx:dx + S, :].reshape(hw, cin)
         for dy in range(3) for dx in range(3)], axis=-1)
    return jnp.dot(patches, w_ref[...],
                   preferred_element_type=jnp.float32) + cb_ref[...]


def _make_resblock_kernel(S, cin, cout, cg1, cg2, has_proj, do_pool):
    hw = S * S
    inv1 = 1.0 / float(hw * cg1)
    inv2 = 1.0 / float(hw * cg2)

    def body(*refs):
        (x_ref, stin_ref, g1_ref, b1_ref, gm1_ref, w1_ref, cb1_ref,
         g2_ref, b2_ref, gm2_ref, w2_ref, cb2_ref) = refs[:12]
        n = 12
        scw_ref = scb_ref = None
        if has_proj:
            scw_ref, scb_ref = refs[12:14]
            n = 14
        o_ref = refs[n]
        pool_ref = stp_ref = None
        if do_pool:
            pool_ref, stp_ref = refs[n + 1], refs[n + 2]
            h_s, st2_s = refs[n + 3], refs[n + 4]
        else:
            h_s, st2_s = refs[n + 1], refs[n + 2]

        p = pl.program_id(1)

        @pl.when(p == 0)
        def _phase_conv1():
            scale, shift = _scale_shift(stin_ref[0, 0:1, :], stin_ref[0, 1:2, :],
                                        gm1_ref, g1_ref, b1_ref, inv1)
            acc = _gn_silu_conv3x3(x_ref[0].astype(jnp.float32), scale, shift,
                                   w1_ref, cb1_ref, S, cin)
            h_s[...] = acc.astype(h_s.dtype)
            ts = jnp.sum(acc, axis=0)
            tss = jnp.sum(acc * acc, axis=0)
            st2_s[...] = jnp.concatenate([ts[None, :], tss[None, :]], axis=0)

        @pl.when(p == 1)
        def _phase_conv2():
            scale, shift = _scale_shift(st2_s[0:1, :], st2_s[1:2, :],
                                        gm2_ref, g2_ref, b2_ref, inv2)
            acc = _gn_silu_conv3x3(h_s[...].astype(jnp.float32), scale, shift,
                                   w2_ref, cb2_ref, S, cout)
            if has_proj:
                acc = acc + (jnp.dot(x_ref[0], scw_ref[...],
                                     preferred_element_type=jnp.float32)
                             + scb_ref[...])
            else:
                acc = acc + x_ref[0].astype(jnp.float32)
            o_ref[0] = acc.T.astype(o_ref.dtype)
            if do_pool:
                v = acc.reshape(S // 2, 2, S // 2, 2, cout)
                pq = 0.25 * (v[:, 0, :, 0, :] + v[:, 0, :, 1, :]
                             + v[:, 1, :, 0, :] + v[:, 1, :, 1, :])
                pf = pq.reshape(hw // 4, cout)
                pool_ref[0] = pf.astype(pool_ref.dtype)
                ps = jnp.sum(pf, axis=0)
                pss = jnp.sum(pf * pf, axis=0)
                stp_ref[0] = jnp.concatenate([ps[None, :], pss[None, :]],
                                             axis=0)

    return body


def _resblock(xf, stats, gn1, gm1, w1, cb1, gn2, gm2, w2, cb2, *, S, groups,
              sc=None, do_pool=False):
    B, hw, cin = xf.shape
    cout = w1.shape[-1]
    f32 = jnp.float32

    def _c(i):
        return lambda b, p: (b,) + (0,) * i

    in_specs = [
        pl.BlockSpec((1, hw, cin), lambda b, p: (b, 0, 0)),
        pl.BlockSpec((1, 2, cin), lambda b, p: (b, 0, 0)),
        pl.BlockSpec((1, cin), lambda b, p: (0, 0)),
        pl.BlockSpec((1, cin), lambda b, p: (0, 0)),
        pl.BlockSpec((cin, cin), lambda b, p: (0, 0)),
        pl.BlockSpec((9 * cin, cout), lambda b, p: (0, 0)),
        pl.BlockSpec((1, cout), lambda b, p: (0, 0)),
        pl.BlockSpec((1, cout), lambda b, p: (0, 0)),
        pl.BlockSpec((1, cout), lambda b, p: (0, 0)),
        pl.BlockSpec((cout, cout), lambda b, p: (0, 0)),
        pl.BlockSpec((9 * cout, cout), lambda b, p: (0, 0)),
        pl.BlockSpec((1, cout), lambda b, p: (0, 0)),
    ]
    args = [xf, stats,
            gn1[0].reshape(1, cin).astype(f32), gn1[1].reshape(1, cin).astype(f32),
            gm1, w1, cb1.reshape(1, cout).astype(f32),
            gn2[0].reshape(1, cout).astype(f32), gn2[1].reshape(1, cout).astype(f32),
            gm2, w2, cb2.reshape(1, cout).astype(f32)]
    if sc is not None:
        in_specs += [pl.BlockSpec((cin, cout), lambda b, p: (0, 0)),
                     pl.BlockSpec((1, cout), lambda b, p: (0, 0))]
        args += [sc[0].astype(jnp.bfloat16), sc[1].reshape(1, cout).astype(f32)]

    out_shapes = [jax.ShapeDtypeStruct((B, cout, hw), f32)]
    out_specs = [pl.BlockSpec((1, cout, hw), lambda b, p: (b, 0, 0))]
    if do_pool:
        out_shapes += [jax.ShapeDtypeStruct((B, hw // 4, cout), jnp.bfloat16),
                       jax.ShapeDtypeStruct((B, 2, cout), f32)]
        out_specs += [pl.BlockSpec((1, hw // 4, cout), lambda b, p: (b, 0, 0)),
                      pl.BlockSpec((1, 2, cout), lambda b, p: (b, 0, 0))]

    return pl.pallas_call(
        _make_resblock_kernel(S, cin, cout, cin // groups, cout // groups,
                              sc is not None, do_pool),
        out_shape=tuple(out_shapes),
        grid=(B, 2),
        in_specs=in_specs,
        out_specs=tuple(out_specs),
        scratch_shapes=[pltpu.VMEM((hw, cout), jnp.bfloat16),
                        pltpu.VMEM((2, cout), f32)],
        compiler_params=pltpu.CompilerParams(
            dimension_semantics=("parallel", "arbitrary"),
            vmem_limit_bytes=_VMEM_LIMIT),
    )(*args)


def _group_mat(c, groups):
    gidx = jnp.arange(c) // (c // groups)
    return (gidx[:, None] == gidx[None, :]).astype(jnp.float32)


def kernel(x, conv_in_w, conv_in_b,
           r0_gn1_gamma, r0_gn1_beta, r0_conv1_w, r0_conv1_b,
           r0_gn2_gamma, r0_gn2_beta, r0_conv2_w, r0_conv2_b,
           r1_gn1_gamma, r1_gn1_beta, r1_conv1_w, r1_conv1_b,
           r1_gn2_gamma, r1_gn2_beta, r1_conv2_w, r1_conv2_b,
           r1_sc_w, r1_sc_b,
           r2_gn1_gamma, r2_gn1_beta, r2_conv1_w, r2_conv1_b,
           r2_gn2_gamma, r2_gn2_beta, r2_conv2_w, r2_conv2_b,
           r2_sc_w, r2_sc_b):
    groups = 32
    B, c0, hr, wr = x.shape
    H, W = hr // 2, wr // 2
    cu = c0 * 4
    xu = (x.reshape(B, c0, H, 2, W, 2).transpose(0, 2, 4, 1, 3, 5)
          .reshape(B, H * W, cu).astype(jnp.bfloat16))

    cin0 = conv_in_w.shape[1]
    stem_out, st = pl.pallas_call(
        _stem_kernel,
        out_shape=(jax.ShapeDtypeStruct((B, H * W, cin0), jnp.bfloat16),
                   jax.ShapeDtypeStruct((B, 2, cin0), jnp.float32)),
        grid=(B,),
        in_specs=[pl.BlockSpec((1, H * W, cu), lambda b: (b, 0, 0)),
                  pl.BlockSpec((cu, cin0), lambda b: (0, 0)),
                  pl.BlockSpec((1, cin0), lambda b: (0, 0))],
        out_specs=(pl.BlockSpec((1, H * W, cin0), lambda b: (b, 0, 0)),
                   pl.BlockSpec((1, 2, cin0), lambda b: (b, 0, 0))),
        compiler_params=pltpu.CompilerParams(
            dimension_semantics=("parallel",),
            vmem_limit_bytes=_VMEM_LIMIT),
    )(xu, conv_in_w.astype(jnp.bfloat16),
      conv_in_b.reshape(1, cin0).astype(jnp.float32))

    blocks = [
        dict(gn1=(r0_gn1_gamma, r0_gn1_beta), w1=r0_conv1_w, b1=r0_conv1_b,
             gn2=(r0_gn2_gamma, r0_gn2_beta), w2=r0_conv2_w, b2=r0_conv2_b,
             sc=None),
        dict(gn1=(r1_gn1_gamma, r1_gn1_beta), w1=r1_conv1_w, b1=r1_conv1_b,
             gn2=(r1_gn2_gamma, r1_gn2_beta), w2=r1_conv2_w, b2=r1_conv2_b,
             sc=(r1_sc_w, r1_sc_b)),
        dict(gn1=(r2_gn1_gamma, r2_gn1_beta), w1=r2_conv1_w, b1=r2_conv1_b,
             gn2=(r2_gn2_gamma, r2_gn2_beta), w2=r2_conv2_w, b2=r2_conv2_b,
             sc=(r2_sc_w, r2_sc_b)),
    ]

    feats = []
    cur, cur_st = stem_out, st
    S = H
    gmats = {}
    for i, bp in enumerate(blocks):
        cin = bp["w1"].shape[2]
        cout = bp["w1"].shape[3]
        for c in (cin, cout):
            if c not in gmats:
                gmats[c] = _group_mat(c, groups)
        w1 = bp["w1"].reshape(9 * cin, cout).astype(jnp.bfloat16)
        w2 = bp["w2"].reshape(9 * cout, cout).astype(jnp.bfloat16)
        last = i == len(blocks) - 1
        out = _resblock(
            cur, cur_st, bp["gn1"], gmats[cin], w1, bp["b1"],
            bp["gn2"], gmats[cout], w2, bp["b2"],
            S=S, groups=groups, sc=bp["sc"], do_pool=not last)
        if last:
            feat = out[0]
        else:
            feat, cur, cur_st = out
        feats.append(feat.reshape(B, cout, S, S))
        S //= 2
    return feats


# speedup vs baseline: 1.4663x; 1.1757x over previous
import jax
import jax.numpy as jnp
from jax import lax
from jax.experimental import pallas as pl
from jax.experimental.pallas import tpu as pltpu

_VMEM_LIMIT = 100 * 1024 * 1024
_EPS = 1e-6


def _stem_kernel(x_ref, w_ref, b_ref, o_ref, st_ref):
    acc = jnp.dot(x_ref[0], w_ref[...],
                  preferred_element_type=jnp.float32) + b_ref[...]
    o_ref[0] = acc.astype(o_ref.dtype)
    s = jnp.sum(acc, axis=0)
    ss = jnp.sum(acc * acc, axis=0)
    st_ref[0] = jnp.concatenate([s[None, :], ss[None, :]], axis=0)


def _scale_shift(s, ss, gm_ref, g_ref, bt_ref, inv):
    mean = jnp.dot(s, gm_ref[...], preferred_element_type=jnp.float32,
                   precision=lax.Precision.HIGHEST) * inv
    ex2 = jnp.dot(ss, gm_ref[...], preferred_element_type=jnp.float32,
                  precision=lax.Precision.HIGHEST) * inv
    var = ex2 - mean * mean
    scale = g_ref[...] * lax.rsqrt(var + _EPS)
    shift = bt_ref[...] - mean * scale
    return scale, shift


def _gn_silu_slab(x, scale, shift, S, cin):
    y = x * scale + shift
    y = y * jax.nn.sigmoid(y)
    yb = y.astype(jnp.bfloat16).reshape(S, S, cin)
    return jnp.pad(yb, ((1, 1), (1, 1), (0, 0)))


def _conv_chunks(yp, w_ref, cb_ref, S, cin, rt):
    for r0 in range(0, S, rt):
        sl = yp[r0:r0 + rt + 2]
        patches = jnp.concatenate(
            [sl[dy:dy + rt, dx:dx + S, :].reshape(rt * S, cin)
             for dy in range(3) for dx in range(3)], axis=-1)
        yield r0, (jnp.dot(patches, w_ref[...],
                           preferred_element_type=jnp.float32) + cb_ref[...])


def _make_resblock_kernel(S, cin, cout, cg1, cg2, has_proj, do_pool):
    hw = S * S
    inv1 = 1.0 / float(hw * cg1)
    inv2 = 1.0 / float(hw * cg2)

    def body(*refs):
        (x_ref, stin_ref, g1_ref, b1_ref, gm1_ref, w1_ref, cb1_ref,
         g2_ref, b2_ref, gm2_ref, w2_ref, cb2_ref) = refs[:12]
        n = 12
        scw_ref = scb_ref = None
        if has_proj:
            scw_ref, scb_ref = refs[12:14]
            n = 14
        o_ref = refs[n]
        pool_ref = stp_ref = None
        if do_pool:
            pool_ref, stp_ref = refs[n + 1], refs[n + 2]
            h_s, st2_s = refs[n + 3], refs[n + 4]
        else:
            h_s, st2_s = refs[n + 1], refs[n + 2]

        p = pl.program_id(1)

        rt = max(2, min(8, 512 // S, S))

        @pl.when(p == 0)
        def _phase_conv1():
            scale, shift = _scale_shift(stin_ref[0, 0:1, :], stin_ref[0, 1:2, :],
                                        gm1_ref, g1_ref, b1_ref, inv1)
            yp = _gn_silu_slab(x_ref[0].astype(jnp.float32), scale, shift,
                               S, cin)
            ts = tss = 0.0
            for r0, acc in _conv_chunks(yp, w1_ref, cb1_ref, S, cin, rt):
                h_s[r0 * S:(r0 + rt) * S, :] = acc.astype(h_s.dtype)
                ts = ts + jnp.sum(acc, axis=0)
                tss = tss + jnp.sum(acc * acc, axis=0)
            st2_s[...] = jnp.concatenate([ts[None, :], tss[None, :]], axis=0)

        @pl.when(p == 1)
        def _phase_conv2():
            scale, shift = _scale_shift(st2_s[0:1, :], st2_s[1:2, :],
                                        gm2_ref, g2_ref, b2_ref, inv2)
            yp = _gn_silu_slab(h_s[...].astype(jnp.float32), scale, shift,
                               S, cout)
            ps = pss = 0.0
            for r0, acc in _conv_chunks(yp, w2_ref, cb2_ref, S, cout, rt):
                a, b = r0 * S, (r0 + rt) * S
                if has_proj:
                    acc = acc + (jnp.dot(x_ref[0, a:b, :], scw_ref[...],
                                         preferred_element_type=jnp.float32)
                                 + scb_ref[...])
                else:
                    acc = acc + x_ref[0, a:b, :].astype(jnp.float32)
                o_ref[0, a:b, :] = acc.astype(o_ref.dtype)
                if do_pool:
                    v = acc.reshape(rt // 2, 2, S // 2, 2, cout)
                    pq = 0.25 * (v[:, 0, :, 0, :] + v[:, 0, :, 1, :]
                                 + v[:, 1, :, 0, :] + v[:, 1, :, 1, :])
                    pf = pq.reshape(rt * S // 4, cout)
                    pool_ref[0, (r0 // 2) * (S // 2):
                             (r0 // 2 + rt // 2) * (S // 2), :] = (
                        pf.astype(pool_ref.dtype))
                    ps = ps + jnp.sum(pf, axis=0)
                    pss = pss + jnp.sum(pf * pf, axis=0)
            if do_pool:
                stp_ref[0] = jnp.concatenate([ps[None, :], pss[None, :]],
                                             axis=0)

    return body


def _resblock(xf, stats, gn1, gm1, w1, cb1, gn2, gm2, w2, cb2, *, S, groups,
              sc=None, do_pool=False):
    B, hw, cin = xf.shape
    cout = w1.shape[-1]
    f32 = jnp.float32

    def _c(i):
        return lambda b, p: (b,) + (0,) * i

    in_specs = [
        pl.BlockSpec((1, hw, cin), lambda b, p: (b, 0, 0)),
        pl.BlockSpec((1, 2, cin), lambda b, p: (b, 0, 0)),
        pl.BlockSpec((1, cin), lambda b, p: (0, 0)),
        pl.BlockSpec((1, cin), lambda b, p: (0, 0)),
        pl.BlockSpec((cin, cin), lambda b, p: (0, 0)),
        pl.BlockSpec((9 * cin, cout), lambda b, p: (0, 0)),
        pl.BlockSpec((1, cout), lambda b, p: (0, 0)),
        pl.BlockSpec((1, cout), lambda b, p: (0, 0)),
        pl.BlockSpec((1, cout), lambda b, p: (0, 0)),
        pl.BlockSpec((cout, cout), lambda b, p: (0, 0)),
        pl.BlockSpec((9 * cout, cout), lambda b, p: (0, 0)),
        pl.BlockSpec((1, cout), lambda b, p: (0, 0)),
    ]
    args = [xf, stats,
            gn1[0].reshape(1, cin).astype(f32), gn1[1].reshape(1, cin).astype(f32),
            gm1, w1, cb1.reshape(1, cout).astype(f32),
            gn2[0].reshape(1, cout).astype(f32), gn2[1].reshape(1, cout).astype(f32),
            gm2, w2, cb2.reshape(1, cout).astype(f32)]
    if sc is not None:
        in_specs += [pl.BlockSpec((cin, cout), lambda b, p: (0, 0)),
                     pl.BlockSpec((1, cout), lambda b, p: (0, 0))]
        args += [sc[0].astype(jnp.bfloat16), sc[1].reshape(1, cout).astype(f32)]

    out_shapes = [jax.ShapeDtypeStruct((B, hw, cout), f32)]
    out_specs = [pl.BlockSpec((1, hw, cout), lambda b, p: (b, 0, 0))]
    if do_pool:
        out_shapes += [jax.ShapeDtypeStruct((B, hw // 4, cout), jnp.bfloat16),
                       jax.ShapeDtypeStruct((B, 2, cout), f32)]
        out_specs += [pl.BlockSpec((1, hw // 4, cout), lambda b, p: (b, 0, 0)),
                      pl.BlockSpec((1, 2, cout), lambda b, p: (b, 0, 0))]

    return pl.pallas_call(
        _make_resblock_kernel(S, cin, cout, cin // groups, cout // groups,
                              sc is not None, do_pool),
        out_shape=tuple(out_shapes),
        grid=(B, 2),
        in_specs=in_specs,
        out_specs=tuple(out_specs),
        scratch_shapes=[pltpu.VMEM((hw, cout), jnp.bfloat16),
                        pltpu.VMEM((2, cout), f32)],
        compiler_params=pltpu.CompilerParams(
            dimension_semantics=("parallel", "arbitrary"),
            vmem_limit_bytes=_VMEM_LIMIT),
    )(*args)


def _group_mat(c, groups):
    gidx = jnp.arange(c) // (c // groups)
    return (gidx[:, None] == gidx[None, :]).astype(jnp.float32)


def kernel(x, conv_in_w, conv_in_b,
           r0_gn1_gamma, r0_gn1_beta, r0_conv1_w, r0_conv1_b,
           r0_gn2_gamma, r0_gn2_beta, r0_conv2_w, r0_conv2_b,
           r1_gn1_gamma, r1_gn1_beta, r1_conv1_w, r1_conv1_b,
           r1_gn2_gamma, r1_gn2_beta, r1_conv2_w, r1_conv2_b,
           r1_sc_w, r1_sc_b,
           r2_gn1_gamma, r2_gn1_beta, r2_conv1_w, r2_conv1_b,
           r2_gn2_gamma, r2_gn2_beta, r2_conv2_w, r2_conv2_b,
           r2_sc_w, r2_sc_b):
    groups = 32
    B, c0, hr, wr = x.shape
    H, W = hr // 2, wr // 2
    cu = c0 * 4
    xu = (x.reshape(B, c0, H, 2, W, 2).transpose(0, 2, 4, 1, 3, 5)
          .reshape(B, H * W, cu).astype(jnp.bfloat16))

    cin0 = conv_in_w.shape[1]
    stem_out, st = pl.pallas_call(
        _stem_kernel,
        out_shape=(jax.ShapeDtypeStruct((B, H * W, cin0), jnp.bfloat16),
                   jax.ShapeDtypeStruct((B, 2, cin0), jnp.float32)),
        grid=(B,),
        in_specs=[pl.BlockSpec((1, H * W, cu), lambda b: (b, 0, 0)),
                  pl.BlockSpec((cu, cin0), lambda b: (0, 0)),
                  pl.BlockSpec((1, cin0), lambda b: (0, 0))],
        out_specs=(pl.BlockSpec((1, H * W, cin0), lambda b: (b, 0, 0)),
                   pl.BlockSpec((1, 2, cin0), lambda b: (b, 0, 0))),
        compiler_params=pltpu.CompilerParams(
            dimension_semantics=("parallel",),
            vmem_limit_bytes=_VMEM_LIMIT),
    )(xu, conv_in_w.astype(jnp.bfloat16),
      conv_in_b.reshape(1, cin0).astype(jnp.float32))

    blocks = [
        dict(gn1=(r0_gn1_gamma, r0_gn1_beta), w1=r0_conv1_w, b1=r0_conv1_b,
             gn2=(r0_gn2_gamma, r0_gn2_beta), w2=r0_conv2_w, b2=r0_conv2_b,
             sc=None),
        dict(gn1=(r1_gn1_gamma, r1_gn1_beta), w1=r1_conv1_w, b1=r1_conv1_b,
             gn2=(r1_gn2_gamma, r1_gn2_beta), w2=r1_conv2_w, b2=r1_conv2_b,
             sc=(r1_sc_w, r1_sc_b)),
        dict(gn1=(r2_gn1_gamma, r2_gn1_beta), w1=r2_conv1_w, b1=r2_conv1_b,
             gn2=(r2_gn2_gamma, r2_gn2_beta), w2=r2_conv2_w, b2=r2_conv2_b,
             sc=(r2_sc_w, r2_sc_b)),
    ]

    feats = []
    cur, cur_st = stem_out, st
    S = H
    gmats = {}
    for i, bp in enumerate(blocks):
        cin = bp["w1"].shape[2]
        cout = bp["w1"].shape[3]
        for c in (cin, cout):
            if c not in gmats:
                gmats[c] = _group_mat(c, groups)
        w1 = bp["w1"].reshape(9 * cin, cout).astype(jnp.bfloat16)
        w2 = bp["w2"].reshape(9 * cout, cout).astype(jnp.bfloat16)
        last = i == len(blocks) - 1
        out = _resblock(
            cur, cur_st, bp["gn1"], gmats[cin], w1, bp["b1"],
            bp["gn2"], gmats[cout], w2, bp["b2"],
            S=S, groups=groups, sc=bp["sc"], do_pool=not last)
        if last:
            feat = out[0]
        else:
            feat, cur, cur_st = out
        feats.append(feat.reshape(B, S, S, cout).transpose(0, 3, 1, 2))
        S //= 2
    return feats
